# Initial kernel scaffold; baseline (speedup 1.0000x reference)
import jax
import jax.numpy as jnp
from jax.experimental import pallas as pl


def kernel(input_ids, token_type_ids, visual_embeds, visual_token_type_ids, word_table, pos_table, tt_table, vtt_table, vpos_table, Wp, bp, ln_w, ln_b):
    raise NotImplementedError("write your pallas kernel here")


# speedup vs baseline: 2.5012x; 2.5012x over previous
import functools

import jax
import jax.numpy as jnp
from jax import lax
from jax.experimental import pallas as pl
from jax.experimental.pallas import tpu as pltpu
from jax.experimental.pallas import tpu_sc as plsc

VOCAB = 30522
HID = 768
B = 64
S = 512
V = 100
VDIM = 2048
EPS = 1e-12

NC = 2
NS = 16
NW = NC * NS
NTOK = B * S
TOK_PER_W = NTOK // NW
CHUNK = 64
NCHUNK = TOK_PER_W // CHUNK

_sc_mesh = plsc.VectorSubcoreMesh(core_axis_name="c", subcore_axis_name="s")


@functools.partial(
    pl.kernel,
    out_type=jax.ShapeDtypeStruct((NTOK, HID), jnp.float32),
    mesh=_sc_mesh,
    scratch_types=[
        pltpu.VMEM((TOK_PER_W,), jnp.int32),
        pltpu.VMEM((2, CHUNK, HID), jnp.float32),
        pltpu.SemaphoreType.DMA,
        pltpu.SemaphoreType.DMA,
    ],
)
def _sc_gather(ids_hbm, table_hbm, out_hbm, idx_v, rows_v, gsem, wsem):
    wid = lax.axis_index("s") * NC + lax.axis_index("c")
    base = wid * TOK_PER_W
    pltpu.sync_copy(ids_hbm.at[pl.ds(base, TOK_PER_W)], idx_v)

    def start_gather(i):
        return pltpu.async_copy(
            table_hbm.at[idx_v.at[pl.ds(i * CHUNK, CHUNK)]],
            rows_v.at[i % 2],
            gsem,
        )

    g = start_gather(0)
    wb = [None, None]
    for i in range(NCHUNK):
        cur = i % 2
        g.wait()
        if i + 1 < NCHUNK:
            if wb[1 - cur] is not None:
                wb[1 - cur].wait()
            g = start_gather(i + 1)
        wb[cur] = pltpu.async_copy(
            rows_v.at[cur],
            out_hbm.at[pl.ds(base + i * CHUNK, CHUNK)],
            wsem,
        )
    wb[(NCHUNK - 2) % 2].wait()
    wb[(NCHUNK - 1) % 2].wait()


def _tc_body(g_ref, pos_ref, ttf_ref, ve_ref, wp_ref, vttf_ref,
             tbias_ref, dtt_ref, vbias_ref, dvtt_ref, lnw_ref, lnb_ref,
             out_ref):
    lnw = lnw_ref[...]
    lnb = lnb_ref[...]

    def layer_norm(x):
        u = jnp.mean(x, axis=-1, keepdims=True)
        d = x - u
        var = jnp.mean(d * d, axis=-1, keepdims=True)
        return lnw * (d * lax.rsqrt(var + EPS)) + lnb

    tx = g_ref[0] + pos_ref[...] + tbias_ref[...] + ttf_ref[0] * dtt_ref[...]
    out_ref[0, :S, :] = layer_norm(tx)

    vis = jnp.dot(ve_ref[0], wp_ref[...], preferred_element_type=jnp.float32)
    vis = vis + vbias_ref[...] + vttf_ref[0] * dvtt_ref[...]
    out_ref[0, S:, :] = layer_norm(vis)


def _tc_fused(g, pos, ttf, ve, wp, vttf, tbias, dtt, vbias, dvtt, lnw, lnb):
    return pl.pallas_call(
        _tc_body,
        grid=(B,),
        in_specs=[
            pl.BlockSpec((1, S, HID), lambda b: (b, 0, 0)),
            pl.BlockSpec((S, HID), lambda b: (0, 0)),
            pl.BlockSpec((1, S, 1), lambda b: (b, 0, 0)),
            pl.BlockSpec((1, V, VDIM), lambda b: (b, 0, 0)),
            pl.BlockSpec((VDIM, HID), lambda b: (0, 0)),
            pl.BlockSpec((1, V, 1), lambda b: (b, 0, 0)),
            pl.BlockSpec((1, HID), lambda b: (0, 0)),
            pl.BlockSpec((1, HID), lambda b: (0, 0)),
            pl.BlockSpec((1, HID), lambda b: (0, 0)),
            pl.BlockSpec((1, HID), lambda b: (0, 0)),
            pl.BlockSpec((1, HID), lambda b: (0, 0)),
            pl.BlockSpec((1, HID), lambda b: (0, 0)),
        ],
        out_specs=pl.BlockSpec((1, S + V, HID), lambda b: (b, 0, 0)),
        out_shape=jax.ShapeDtypeStruct((B, S + V, HID), jnp.float32),
    )(g, pos, ttf, ve, wp, vttf, tbias, dtt, vbias, dvtt, lnw, lnb)


def kernel(input_ids, token_type_ids, visual_embeds, visual_token_type_ids,
           word_table, pos_table, tt_table, vtt_table, vpos_table, Wp, bp,
           ln_w, ln_b):
    ids = input_ids.reshape(-1).astype(jnp.int32)
    g = _sc_gather(ids, word_table)
    g = g.reshape(B, S, HID)

    ttf = (token_type_ids != 0).astype(jnp.float32).reshape(B, S, 1)
    vttf = (visual_token_type_ids != 0).astype(jnp.float32).reshape(B, V, 1)
    tbias = tt_table[0].reshape(1, HID)
    dtt = (tt_table[1] - tt_table[0]).reshape(1, HID)
    vbias = (bp + vpos_table[0] + vtt_table[0]).reshape(1, HID)
    dvtt = (vtt_table[1] - vtt_table[0]).reshape(1, HID)

    return _tc_fused(g, pos_table, ttf, visual_embeds, Wp, vttf,
                     tbias, dtt, vbias, dvtt,
                     ln_w.reshape(1, HID), ln_b.reshape(1, HID))


# speedup vs baseline: 2.6859x; 1.0739x over previous
import functools

import jax
import jax.numpy as jnp
from jax import lax
from jax.experimental import pallas as pl
from jax.experimental.pallas import tpu as pltpu
from jax.experimental.pallas import tpu_sc as plsc

VOCAB = 30522
HID = 768
B = 64
S = 512
V = 100
VDIM = 2048
EPS = 1e-12

NC = 2
NS = 16
NW = NC * NS
NTOK = B * S
TOK_PER_W = NTOK // NW
CHUNK = 64
NCHUNK = TOK_PER_W // CHUNK

_sc_mesh = plsc.VectorSubcoreMesh(core_axis_name="c", subcore_axis_name="s")


@functools.partial(
    pl.kernel,
    out_type=jax.ShapeDtypeStruct((NTOK, HID), jnp.float32),
    mesh=_sc_mesh,
    scratch_types=[
        pltpu.VMEM((TOK_PER_W,), jnp.int32),
        pltpu.VMEM((2, CHUNK, HID), jnp.float32),
        pltpu.SemaphoreType.DMA,
        pltpu.SemaphoreType.DMA,
    ],
)
def _sc_gather(ids_hbm, table_hbm, out_hbm, idx_v, rows_v, gsem, wsem):
    wid = lax.axis_index("s") * NC + lax.axis_index("c")
    base = wid * TOK_PER_W
    pltpu.sync_copy(ids_hbm.at[pl.ds(base, TOK_PER_W)], idx_v)

    def start_gather(i):
        return pltpu.async_copy(
            table_hbm.at[idx_v.at[pl.ds(i * CHUNK, CHUNK)]],
            rows_v.at[i % 2],
            gsem,
        )

    g = start_gather(0)
    wb = [None, None]
    for i in range(NCHUNK):
        cur = i % 2
        g.wait()
        if i + 1 < NCHUNK:
            if wb[1 - cur] is not None:
                wb[1 - cur].wait()
            g = start_gather(i + 1)
        wb[cur] = pltpu.async_copy(
            rows_v.at[cur],
            out_hbm.at[pl.ds(base + i * CHUNK, CHUNK)],
            wsem,
        )
    wb[(NCHUNK - 2) % 2].wait()
    wb[(NCHUNK - 1) % 2].wait()


BB = 4


def _tc_body(g_ref, pos_ref, ttf_ref, ve_ref, wp_ref, vttf_ref,
             tbias_ref, dtt_ref, vbias_ref, dvtt_ref, lnw_ref, lnb_ref,
             out_ref):
    lnw = lnw_ref[...]
    lnb = lnb_ref[...]

    def layer_norm(x):
        u = jnp.mean(x, axis=-1, keepdims=True)
        d = x - u
        var = jnp.mean(d * d, axis=-1, keepdims=True)
        return lnw * (d * lax.rsqrt(var + EPS)) + lnb

    tx = g_ref[...] + pos_ref[...] + tbias_ref[...] + ttf_ref[...] * dtt_ref[...]
    out_ref[:, :S, :] = layer_norm(tx)

    ve2 = ve_ref[...].reshape(BB * V, VDIM)
    vis = jnp.dot(ve2, wp_ref[...], preferred_element_type=jnp.float32)
    vis = vis.reshape(BB, V, HID)
    vis = vis + vbias_ref[...] + vttf_ref[...] * dvtt_ref[...]
    out_ref[:, S:, :] = layer_norm(vis)


def _tc_fused(g, pos, ttf, ve, wp, vttf, tbias, dtt, vbias, dvtt, lnw, lnb):
    return pl.pallas_call(
        _tc_body,
        grid=(B // BB,),
        in_specs=[
            pl.BlockSpec((BB, S, HID), lambda b: (b, 0, 0)),
            pl.BlockSpec((1, S, HID), lambda b: (0, 0, 0)),
            pl.BlockSpec((BB, S, 1), lambda b: (b, 0, 0)),
            pl.BlockSpec((BB, V, VDIM), lambda b: (b, 0, 0)),
            pl.BlockSpec((VDIM, HID), lambda b: (0, 0)),
            pl.BlockSpec((BB, V, 1), lambda b: (b, 0, 0)),
            pl.BlockSpec((1, 1, HID), lambda b: (0, 0, 0)),
            pl.BlockSpec((1, 1, HID), lambda b: (0, 0, 0)),
            pl.BlockSpec((1, 1, HID), lambda b: (0, 0, 0)),
            pl.BlockSpec((1, 1, HID), lambda b: (0, 0, 0)),
            pl.BlockSpec((1, 1, HID), lambda b: (0, 0, 0)),
            pl.BlockSpec((1, 1, HID), lambda b: (0, 0, 0)),
        ],
        out_specs=pl.BlockSpec((BB, S + V, HID), lambda b: (b, 0, 0)),
        out_shape=jax.ShapeDtypeStruct((B, S + V, HID), jnp.float32),
    )(g, pos, ttf, ve, wp, vttf, tbias, dtt, vbias, dvtt, lnw, lnb)


def kernel(input_ids, token_type_ids, visual_embeds, visual_token_type_ids,
           word_table, pos_table, tt_table, vtt_table, vpos_table, Wp, bp,
           ln_w, ln_b):
    ids = input_ids.reshape(-1).astype(jnp.int32)
    g = _sc_gather(ids, word_table)
    g = g.reshape(B, S, HID)

    ttf = (token_type_ids != 0).astype(jnp.float32).reshape(B, S, 1)
    vttf = (visual_token_type_ids != 0).astype(jnp.float32).reshape(B, V, 1)
    tbias = tt_table[0].reshape(1, 1, HID)
    dtt = (tt_table[1] - tt_table[0]).reshape(1, 1, HID)
    vbias = (bp + vpos_table[0] + vtt_table[0]).reshape(1, 1, HID)
    dvtt = (vtt_table[1] - vtt_table[0]).reshape(1, 1, HID)

    return _tc_fused(g, pos_table.reshape(1, S, HID), ttf, visual_embeds, Wp,
                     vttf, tbias, dtt, vbias, dvtt,
                     ln_w.reshape(1, 1, HID), ln_b.reshape(1, 1, HID))
